# baseline (speedup 1.0000x reference)
import functools

import jax
import jax.numpy as jnp
from jax import lax
from jax.experimental import pallas as pl
from jax.experimental.pallas import tpu as pltpu, tpu_sc as plsc

_NC = 2
_NS = 16
_L = 16
_NW = _NC * _NS


def _onehot_body(n_tokens, num_classes, y_hbm, out1_hbm, out2_hbm, y_v, oh_v):
    tpw = n_tokens // _NW
    wid = lax.axis_index("s") * _NC + lax.axis_index("c")
    base = wid * tpw

    pltpu.sync_copy(y_hbm.at[pl.ds(base, tpw)], y_v)

    zeros = jnp.zeros((_L,), jnp.float32)
    ones = jnp.ones((_L,), jnp.float32)
    lane = lax.iota(jnp.int32, _L)

    def group(g, c):
        row0 = g * _L
        for j in range(_L):
            oh_v[pl.ds((row0 + j) * num_classes, _L)] = zeros
        labels = y_v[pl.ds(row0, _L)]
        plsc.store_scatter(oh_v, [(row0 + lane) * num_classes + labels], ones)
        return c

    lax.fori_loop(0, tpw // _L, group, None)

    pltpu.sync_copy(oh_v, out1_hbm.at[pl.ds(base * num_classes, tpw * num_classes)])
    pltpu.sync_copy(oh_v, out2_hbm.at[pl.ds(base * num_classes, tpw * num_classes)])


def kernel(functional_samples, x, expected_logbeta, y, mollify, mixer, temperature):
    num_classes = functional_samples.shape[1]
    n = y.shape[0]
    tpw = n // _NW
    y32 = y.astype(jnp.int32)

    mesh = plsc.VectorSubcoreMesh(
        core_axis_name="c", subcore_axis_name="s",
        num_cores=_NC, num_subcores=_NS,
    )
    oh_shape = jax.ShapeDtypeStruct((n * num_classes,), jnp.float32)
    sc_call = pl.kernel(
        functools.partial(_onehot_body, n, num_classes),
        out_type=(oh_shape, oh_shape),
        mesh=mesh,
        scratch_types=[
            pltpu.VMEM((tpw,), jnp.int32),
            pltpu.VMEM((tpw * num_classes,), jnp.float32),
        ],
        compiler_params=pltpu.CompilerParams(needs_layout_passes=False),
    )
    out1, out2 = sc_call(y32)
    zero = jnp.zeros((), dtype=jnp.float32)
    return (out1.reshape(n, num_classes), zero, out2.reshape(n, num_classes))


# speedup vs baseline: 1.1502x; 1.1502x over previous
import functools

import jax
import jax.numpy as jnp
from jax import lax
from jax.experimental import pallas as pl
from jax.experimental.pallas import tpu as pltpu, tpu_sc as plsc

_NC = 2
_NS = 16
_L = 16
_NW = _NC * _NS


def _onehot_body(n_tokens, num_classes, y_hbm, out_hbm, y_v, oh_v):
    tpw = n_tokens // _NW
    wid = lax.axis_index("s") * _NC + lax.axis_index("c")
    base = wid * tpw

    pltpu.sync_copy(y_hbm.at[pl.ds(base, tpw)], y_v)

    zeros = jnp.zeros((_L,), jnp.float32)
    ones = jnp.ones((_L,), jnp.float32)
    lane = lax.iota(jnp.int32, _L)

    def group(g, c):
        row0 = g * _L
        for j in range(_L):
            oh_v[pl.ds((row0 + j) * num_classes, _L)] = zeros
        labels = y_v[pl.ds(row0, _L)]
        plsc.store_scatter(oh_v, [(row0 + lane) * num_classes + labels], ones)
        return c

    lax.fori_loop(0, tpw // _L, group, None)

    pltpu.sync_copy(oh_v, out_hbm.at[pl.ds(base * num_classes, tpw * num_classes)])


def kernel(functional_samples, x, expected_logbeta, y, mollify, mixer, temperature):
    num_classes = functional_samples.shape[1]
    n = y.shape[0]
    tpw = n // _NW
    y32 = y.astype(jnp.int32)

    mesh = plsc.VectorSubcoreMesh(
        core_axis_name="c", subcore_axis_name="s",
        num_cores=_NC, num_subcores=_NS,
    )
    oh_shape = jax.ShapeDtypeStruct((n * num_classes,), jnp.float32)
    sc_call = pl.kernel(
        functools.partial(_onehot_body, n, num_classes),
        out_type=oh_shape,
        mesh=mesh,
        scratch_types=[
            pltpu.VMEM((tpw,), jnp.int32),
            pltpu.VMEM((tpw * num_classes,), jnp.float32),
        ],
        compiler_params=pltpu.CompilerParams(needs_layout_passes=False),
    )
    out = sc_call(y32).reshape(n, num_classes)
    zero = jnp.zeros((), dtype=jnp.float32)
    return (out, zero, out)
